# baseline (speedup 1.0000x reference)
import jax
import jax.numpy as jnp
from jax.experimental import pallas as pl

_PLACEHOLDER_TOKEN = 500
_ROW_BLOCK = 1024


def _where_body(tok_ref, emb_ref, vec_ref, out_ref):
    mask = tok_ref[...] == _PLACEHOLDER_TOKEN
    out_ref[...] = jnp.where(mask, vec_ref[...], emb_ref[...])


def kernel(tokenized_text, embedded_text, placeholder_embedding):
    b, n, d = embedded_text.shape
    rows = b * n
    grid = rows // _ROW_BLOCK
    emb = embedded_text.reshape(rows, d)
    tok = tokenized_text.reshape(rows, 1)
    out = pl.pallas_call(
        _where_body,
        grid=(grid,),
        in_specs=[
            pl.BlockSpec((_ROW_BLOCK, 1), lambda i: (i, 0)),
            pl.BlockSpec((_ROW_BLOCK, d), lambda i: (i, 0)),
            pl.BlockSpec((1, d), lambda i: (0, 0)),
        ],
        out_specs=pl.BlockSpec((_ROW_BLOCK, d), lambda i: (i, 0)),
        out_shape=jax.ShapeDtypeStruct((rows, d), embedded_text.dtype),
    )(tok, emb, placeholder_embedding)
    return out.reshape(b, n, d)


# speedup vs baseline: 1.0300x; 1.0300x over previous
import jax
import jax.numpy as jnp
from jax.experimental import pallas as pl

_PLACEHOLDER_TOKEN = 500
_ROW_BLOCK = 2048


def _where_body(tok_ref, emb_ref, vec_ref, out_ref):
    mask = tok_ref[...] == _PLACEHOLDER_TOKEN
    out_ref[...] = jnp.where(mask, vec_ref[...], emb_ref[...])


def kernel(tokenized_text, embedded_text, placeholder_embedding):
    b, n, d = embedded_text.shape
    rows = b * n
    grid = rows // _ROW_BLOCK
    emb = embedded_text.reshape(rows, d)
    tok = tokenized_text.reshape(rows, 1)
    out = pl.pallas_call(
        _where_body,
        grid=(grid,),
        in_specs=[
            pl.BlockSpec((_ROW_BLOCK, 1), lambda i: (i, 0)),
            pl.BlockSpec((_ROW_BLOCK, d), lambda i: (i, 0)),
            pl.BlockSpec((1, d), lambda i: (0, 0)),
        ],
        out_specs=pl.BlockSpec((_ROW_BLOCK, d), lambda i: (i, 0)),
        out_shape=jax.ShapeDtypeStruct((rows, d), embedded_text.dtype),
    )(tok, emb, placeholder_embedding)
    return out.reshape(b, n, d)
